# Initial kernel scaffold; baseline (speedup 1.0000x reference)
import jax
import jax.numpy as jnp
from jax.experimental import pallas as pl


def kernel(x, edge_attr, edge_index, batch, W1, b1, g1, be1, W2, b2, g2, be2, W3, b3, g3, be3, lw1, lb1, lw2, lb2):
    raise NotImplementedError("write your pallas kernel here")


# speedup vs baseline: 10.7940x; 10.7940x over previous
import functools

import jax
import jax.numpy as jnp
from jax import lax
from jax.experimental import pallas as pl
from jax.experimental.pallas import tpu as pltpu
from jax.experimental.pallas import tpu_sc as plsc

N = 10000
D = 128
G = 64
E = 320000
NC, NS = 2, 16
NW = NC * NS
CH = 128
K = 79
EPAD = NW * K * CH
NP = N + 112
RPS = NP // NS
DEGW = 16

_f32 = jnp.float32


def _sc_mesh():
    return plsc.VectorSubcoreMesh(core_axis_name="c", subcore_axis_name="s")


def _sc_degree(dst_idx, ones_rows, zrows):
    @functools.partial(
        pl.kernel,
        out_type=jax.ShapeDtypeStruct((NC, NP, D), _f32),
        mesh=_sc_mesh(),
        scratch_types=[
            pltpu.VMEM((K, CH), jnp.int32),
            pltpu.VMEM((CH, D), _f32),
            pltpu.VMEM_SHARED((NP, D), _f32),
        ],
    )
    def k(dst_hbm, ones_hbm, z_hbm, out_hbm, dst_v, ones_v, accd):
        c = lax.axis_index("c")
        s = lax.axis_index("s")
        w = c * NS + s
        pltpu.sync_copy(z_hbm.at[pl.ds(s * RPS, RPS)], accd.at[pl.ds(s * RPS, RPS)])
        pltpu.sync_copy(dst_hbm.at[w], dst_v)
        pltpu.sync_copy(ones_hbm, ones_v)
        plsc.subcore_barrier()

        def body(j, carry):
            pltpu.sync_copy(ones_v, accd.at[dst_v.at[j]], add=True)
            return carry

        lax.fori_loop(0, K, body, 0)
        plsc.subcore_barrier()
        pltpu.sync_copy(accd.at[pl.ds(s * RPS, RPS)],
                        out_hbm.at[c, pl.ds(s * RPS, RPS)])

    return k(dst_idx, ones_rows, zrows)


def _sc_scatter(y, src_idx, dst_idx, zrows):
    @functools.partial(
        pl.kernel,
        out_type=jax.ShapeDtypeStruct((NC, NP, D), _f32),
        mesh=_sc_mesh(),
        scratch_types=[
            pltpu.VMEM((K, CH), jnp.int32),
            pltpu.VMEM((K, CH), jnp.int32),
            pltpu.VMEM((CH, D), _f32),
            pltpu.VMEM_SHARED((NP, D), _f32),
            pltpu.SemaphoreType.DMA,
        ],
    )
    def k(y_hbm, src_hbm, dst_hbm, z_hbm, out_hbm, src_v, dst_v, rows_v, acc, sem):
        c = lax.axis_index("c")
        s = lax.axis_index("s")
        w = c * NS + s
        pltpu.sync_copy(z_hbm.at[pl.ds(s * RPS, RPS)], acc.at[pl.ds(s * RPS, RPS)])
        pltpu.sync_copy(src_hbm.at[w], src_v)
        pltpu.sync_copy(dst_hbm.at[w], dst_v)
        plsc.subcore_barrier()

        def body(j, carry):
            pltpu.async_copy(y_hbm.at[src_v.at[j]], rows_v, sem).wait()
            pltpu.sync_copy(rows_v, acc.at[dst_v.at[j]], add=True)
            return carry

        lax.fori_loop(0, K, body, 0)
        plsc.subcore_barrier()
        pltpu.sync_copy(acc.at[pl.ds(s * RPS, RPS)],
                        out_hbm.at[c, pl.ds(s * RPS, RPS)])

    return k(y, src_idx, dst_idx, zrows)


def _tc_stage1(degp, x, W1):
    def body(degp_ref, x_ref, w_ref, dinv_ref, y_ref):
        dp = degp_ref[...]
        deg = dp[0, :N, :1] + dp[1, :N, :1] + 1.0
        dinv = lax.rsqrt(deg)
        dinv_ref[...] = dinv
        xw = jnp.dot(x_ref[...], w_ref[...], preferred_element_type=_f32)
        y_ref[...] = xw * dinv

    return pl.pallas_call(
        body,
        out_shape=(jax.ShapeDtypeStruct((N, 1), _f32),
                   jax.ShapeDtypeStruct((N, D), _f32)),
    )(degp, x, W1)


def _bn_relu(accp_ref, y_ref, dinv_ref, b_ref, g_ref, be_ref):
    ap = accp_ref[...]
    acc = ap[0, :N, :] + ap[1, :N, :] + y_ref[...]
    r = jnp.maximum(acc * dinv_ref[...] + b_ref[...], 0.0)
    m = jnp.mean(r, axis=0, keepdims=True)
    v = jnp.mean(r * r, axis=0, keepdims=True) - m * m
    return (r - m) * lax.rsqrt(v + 1e-5) * g_ref[...] + be_ref[...]


def _tc_layer(accp, y, dinv, b, g, be, Wn):
    def body(accp_ref, y_ref, dinv_ref, b_ref, g_ref, be_ref, w_ref, yn_ref):
        h = _bn_relu(accp_ref, y_ref, dinv_ref, b_ref, g_ref, be_ref)
        yn_ref[...] = jnp.dot(h, w_ref[...], preferred_element_type=_f32) * dinv_ref[...]

    return pl.pallas_call(
        body,
        out_shape=jax.ShapeDtypeStruct((N, D), _f32),
    )(accp, y, dinv, b, g, be, Wn)


def _tc_final(accp, y, dinv, b, g, be, batch2d, lw1, lb1, lw2, lb2):
    def body(accp_ref, y_ref, dinv_ref, b_ref, g_ref, be_ref, bat_ref,
             lw1_ref, lb1_ref, lw2_ref, lb2_ref, out_ref, pooled_ref):
        h = _bn_relu(accp_ref, y_ref, dinv_ref, b_ref, g_ref, be_ref)
        bat = bat_ref[...]
        neg = _f32(-jnp.inf)

        def pool_one(gi, carry):
            row = jnp.max(jnp.where(bat == gi, h, neg), axis=0, keepdims=True)
            pooled_ref[pl.ds(gi, 1), :] = row
            return carry

        lax.fori_loop(0, G, pool_one, 0)
        pooled = pooled_ref[...]
        h2 = jnp.maximum(
            jnp.dot(pooled, lw1_ref[...], preferred_element_type=_f32) + lb1_ref[...],
            0.0)
        out_ref[...] = jnp.dot(h2, lw2_ref[...], preferred_element_type=_f32) + lb2_ref[...]

    return pl.pallas_call(
        body,
        out_shape=jax.ShapeDtypeStruct((G, 1), _f32),
        scratch_shapes=[pltpu.VMEM((G, D), _f32)],
    )(accp, y, dinv, b, g, be, batch2d, lw1, lb1, lw2, lb2)


def kernel(x, edge_attr, edge_index, batch,
           W1, b1, g1, be1, W2, b2, g2, be2, W3, b3, g3, be3,
           lw1, lb1, lw2, lb2):
    pad = EPAD - E
    srcp = jnp.concatenate(
        [edge_index[0], jnp.zeros((pad,), jnp.int32)]).reshape(NW, K, CH)
    dstp = jnp.concatenate(
        [edge_index[1], jnp.full((pad,), N, jnp.int32)]).reshape(NW, K, CH)
    zrows = jnp.zeros((NP, D), _f32)
    batch2d = batch.reshape(N, 1)
    b1r, g1r, be1r = b1.reshape(1, D), g1.reshape(1, D), be1.reshape(1, D)
    b2r, g2r, be2r = b2.reshape(1, D), g2.reshape(1, D), be2.reshape(1, D)
    b3r, g3r, be3r = b3.reshape(1, D), g3.reshape(1, D), be3.reshape(1, D)
    lb1r = lb1.reshape(1, D // 2)
    lb2r = lb2.reshape(1, 1)

    degp = _sc_degree(dstp, jnp.ones((CH, D), _f32), zrows)
    dinv, y1 = _tc_stage1(degp, x, W1)
    acc1 = _sc_scatter(y1, srcp, dstp, zrows)
    y2 = _tc_layer(acc1, y1, dinv, b1r, g1r, be1r, W2)
    acc2 = _sc_scatter(y2, srcp, dstp, zrows)
    y3 = _tc_layer(acc2, y2, dinv, b2r, g2r, be2r, W3)
    acc3 = _sc_scatter(y3, srcp, dstp, zrows)
    return _tc_final(acc3, y3, dinv, b3r, g3r, be3r, batch2d,
                     lw1, lb1r, lw2, lb2r)
